# Initial kernel scaffold; baseline (speedup 1.0000x reference)
import jax
import jax.numpy as jnp
from jax.experimental import pallas as pl


def kernel(x, shared_w1, shared_w3, shared_w2, routed_w1, routed_w3, routed_w2, gate_w, expert_bias):
    raise NotImplementedError("write your pallas kernel here")


# speedup vs baseline: 2.6424x; 2.6424x over previous
import functools
import jax
import jax.numpy as jnp
from jax.experimental import pallas as pl

TB = 256


def _silu(t):
    return t * jax.nn.sigmoid(t)


def _moe_block(x_ref, w1s_ref, w3s_ref, w2s_ref, rw1_ref, rw3_ref, rw2_ref,
               gw_ref, bias_ref, out_ref, idx_ref, *, n_routed):
    x = x_ref[...]

    scores = jax.nn.sigmoid(
        jnp.dot(x, gw_ref[...], preferred_element_type=jnp.float32))
    sel = scores + bias_ref[...]
    e_iota = jax.lax.broadcasted_iota(jnp.int32, sel.shape, 1)

    v0 = jnp.max(sel, axis=1, keepdims=True)
    idx0 = jnp.min(jnp.where(sel == v0, e_iota, n_routed), axis=1)
    sel2 = jnp.where(e_iota == idx0[:, None], -jnp.inf, sel)
    v1 = jnp.max(sel2, axis=1, keepdims=True)
    idx1 = jnp.min(jnp.where(sel2 == v1, e_iota, n_routed), axis=1)

    s0 = jnp.sum(jnp.where(e_iota == idx0[:, None], scores, 0.0), axis=1)
    s1 = jnp.sum(jnp.where(e_iota == idx1[:, None], scores, 0.0), axis=1)
    denom = s0 + s1
    w0 = s0 / denom
    w1 = s1 / denom

    idx_ref[:, 0] = idx0
    idx_ref[:, 1] = idx1

    h = _silu(jnp.dot(x, w1s_ref[...], preferred_element_type=jnp.float32))
    h = h * jnp.dot(x, w3s_ref[...], preferred_element_type=jnp.float32)
    acc = jnp.dot(h, w2s_ref[...], preferred_element_type=jnp.float32)

    for e in range(n_routed):
        coef = w0 * (idx0 == e) + w1 * (idx1 == e)
        he = _silu(jnp.dot(x, rw1_ref[e], preferred_element_type=jnp.float32))
        he = he * jnp.dot(x, rw3_ref[e], preferred_element_type=jnp.float32)
        acc = acc + jnp.dot(coef[:, None] * he, rw2_ref[e],
                            preferred_element_type=jnp.float32)

    out_ref[...] = acc


def kernel(x, shared_w1, shared_w3, shared_w2, routed_w1, routed_w3, routed_w2,
           gate_w, expert_bias):
    b, s, d = x.shape
    t = b * s
    f = shared_w1.shape[-1]
    n_routed = routed_w1.shape[0]
    xf = x.reshape(t, d)

    w1s = jnp.concatenate(list(shared_w1), axis=1)
    w3s = jnp.concatenate(list(shared_w3), axis=1)
    w2s = jnp.concatenate(list(shared_w2), axis=0)

    grid = (t // TB,)
    const = lambda i: (0, 0)
    const3 = lambda i: (0, 0, 0)

    out, idx = pl.pallas_call(
        functools.partial(_moe_block, n_routed=n_routed),
        grid=grid,
        in_specs=[
            pl.BlockSpec((TB, d), lambda i: (i, 0)),
            pl.BlockSpec(w1s.shape, const),
            pl.BlockSpec(w3s.shape, const),
            pl.BlockSpec(w2s.shape, const),
            pl.BlockSpec(routed_w1.shape, const3),
            pl.BlockSpec(routed_w3.shape, const3),
            pl.BlockSpec(routed_w2.shape, const3),
            pl.BlockSpec(gate_w.shape, const),
            pl.BlockSpec(expert_bias.shape, lambda i: (0,)),
        ],
        out_specs=[
            pl.BlockSpec((TB, d), lambda i: (i, 0)),
            pl.BlockSpec((TB, 2), lambda i: (i, 0)),
        ],
        out_shape=[
            jax.ShapeDtypeStruct((t, d), jnp.float32),
            jax.ShapeDtypeStruct((t, 2), jnp.int32),
        ],
    )(xf, w1s, w3s, w2s, routed_w1, routed_w3, routed_w2, gate_w, expert_bias)

    return out.reshape(b, s, d), idx.reshape(b, s, 2)


# speedup vs baseline: 2.6740x; 1.0119x over previous
import functools
import jax
import jax.numpy as jnp
from jax.experimental import pallas as pl

TB = 256


def _silu(t):
    return t * jax.nn.sigmoid(t)


def _moe_block(x_ref, w1s_ref, w3s_ref, w2s_ref, rw1_ref, rw3_ref, rw2_ref,
               gw_ref, bias_ref, out_ref, idx_ref, *, n_routed):
    x = x_ref[...]

    scores = jax.nn.sigmoid(
        jnp.dot(x, gw_ref[...], preferred_element_type=jnp.float32))
    sel = scores + bias_ref[...]
    e_iota = jax.lax.broadcasted_iota(jnp.int32, sel.shape, 1)

    v0 = jnp.max(sel, axis=1, keepdims=True)
    idx0 = jnp.min(jnp.where(sel == v0, e_iota, n_routed), axis=1)
    sel2 = jnp.where(e_iota == idx0[:, None], -jnp.inf, sel)
    v1 = jnp.max(sel2, axis=1, keepdims=True)
    idx1 = jnp.min(jnp.where(sel2 == v1, e_iota, n_routed), axis=1)

    s0 = jnp.sum(jnp.where(e_iota == idx0[:, None], scores, 0.0), axis=1)
    s1 = jnp.sum(jnp.where(e_iota == idx1[:, None], scores, 0.0), axis=1)
    denom = s0 + s1
    w0 = s0 / denom
    w1 = s1 / denom

    idx_ref[:, 0] = idx0
    idx_ref[:, 1] = idx1

    xb = x.astype(jnp.bfloat16)
    h = _silu(jnp.dot(xb, w1s_ref[...].astype(jnp.bfloat16),
                      preferred_element_type=jnp.float32))
    h = h * jnp.dot(xb, w3s_ref[...].astype(jnp.bfloat16),
                    preferred_element_type=jnp.float32)
    acc = jnp.dot(h.astype(jnp.bfloat16), w2s_ref[...].astype(jnp.bfloat16),
                  preferred_element_type=jnp.float32)

    for e in range(n_routed):
        coef = w0 * (idx0 == e) + w1 * (idx1 == e)
        he = _silu(jnp.dot(xb, rw1_ref[e].astype(jnp.bfloat16),
                           preferred_element_type=jnp.float32))
        he = he * jnp.dot(xb, rw3_ref[e].astype(jnp.bfloat16),
                          preferred_element_type=jnp.float32)
        acc = acc + jnp.dot((coef[:, None] * he).astype(jnp.bfloat16),
                            rw2_ref[e].astype(jnp.bfloat16),
                            preferred_element_type=jnp.float32)

    out_ref[...] = acc


def kernel(x, shared_w1, shared_w3, shared_w2, routed_w1, routed_w3, routed_w2,
           gate_w, expert_bias):
    b, s, d = x.shape
    t = b * s
    f = shared_w1.shape[-1]
    n_routed = routed_w1.shape[0]
    xf = x.reshape(t, d)

    w1s = jnp.concatenate(list(shared_w1), axis=1)
    w3s = jnp.concatenate(list(shared_w3), axis=1)
    w2s = jnp.concatenate(list(shared_w2), axis=0)

    grid = (t // TB,)
    const = lambda i: (0, 0)
    const3 = lambda i: (0, 0, 0)

    out, idx = pl.pallas_call(
        functools.partial(_moe_block, n_routed=n_routed),
        grid=grid,
        in_specs=[
            pl.BlockSpec((TB, d), lambda i: (i, 0)),
            pl.BlockSpec(w1s.shape, const),
            pl.BlockSpec(w3s.shape, const),
            pl.BlockSpec(w2s.shape, const),
            pl.BlockSpec(routed_w1.shape, const3),
            pl.BlockSpec(routed_w3.shape, const3),
            pl.BlockSpec(routed_w2.shape, const3),
            pl.BlockSpec(gate_w.shape, const),
            pl.BlockSpec(expert_bias.shape, lambda i: (0,)),
        ],
        out_specs=[
            pl.BlockSpec((TB, d), lambda i: (i, 0)),
            pl.BlockSpec((TB, 2), lambda i: (i, 0)),
        ],
        out_shape=[
            jax.ShapeDtypeStruct((t, d), jnp.float32),
            jax.ShapeDtypeStruct((t, 2), jnp.int32),
        ],
    )(xf, w1s, w3s, w2s, routed_w1, routed_w3, routed_w2, gate_w, expert_bias)

    return out.reshape(b, s, d), idx.reshape(b, s, 2)


# speedup vs baseline: 2.7606x; 1.0324x over previous
import functools
import jax
import jax.numpy as jnp
from jax.experimental import pallas as pl

TB = 1024


def _silu(t):
    return t * jax.nn.sigmoid(t)


def _moe_block(x_ref, w1s_ref, w3s_ref, w2s_ref, rw1_ref, rw3_ref, rw2_ref,
               gw_ref, bias_ref, out_ref, idx_ref, *, n_routed):
    x = x_ref[...]

    scores = jax.nn.sigmoid(
        jnp.dot(x, gw_ref[...], preferred_element_type=jnp.float32))
    sel = scores + bias_ref[...]
    e_iota = jax.lax.broadcasted_iota(jnp.int32, sel.shape, 1)

    v0 = jnp.max(sel, axis=1, keepdims=True)
    idx0 = jnp.min(jnp.where(sel == v0, e_iota, n_routed), axis=1)
    sel2 = jnp.where(e_iota == idx0[:, None], -jnp.inf, sel)
    v1 = jnp.max(sel2, axis=1, keepdims=True)
    idx1 = jnp.min(jnp.where(sel2 == v1, e_iota, n_routed), axis=1)

    s0 = jnp.sum(jnp.where(e_iota == idx0[:, None], scores, 0.0), axis=1)
    s1 = jnp.sum(jnp.where(e_iota == idx1[:, None], scores, 0.0), axis=1)
    denom = s0 + s1
    w0 = s0 / denom
    w1 = s1 / denom

    idx_ref[:, 0] = idx0
    idx_ref[:, 1] = idx1

    xb = x.astype(jnp.bfloat16)
    h = _silu(jnp.dot(xb, w1s_ref[...].astype(jnp.bfloat16),
                      preferred_element_type=jnp.float32))
    h = h * jnp.dot(xb, w3s_ref[...].astype(jnp.bfloat16),
                    preferred_element_type=jnp.float32)
    acc = jnp.dot(h.astype(jnp.bfloat16), w2s_ref[...].astype(jnp.bfloat16),
                  preferred_element_type=jnp.float32)

    for e in range(n_routed):
        coef = w0 * (idx0 == e) + w1 * (idx1 == e)
        he = _silu(jnp.dot(xb, rw1_ref[e].astype(jnp.bfloat16),
                           preferred_element_type=jnp.float32))
        he = he * jnp.dot(xb, rw3_ref[e].astype(jnp.bfloat16),
                          preferred_element_type=jnp.float32)
        acc = acc + jnp.dot((coef[:, None] * he).astype(jnp.bfloat16),
                            rw2_ref[e].astype(jnp.bfloat16),
                            preferred_element_type=jnp.float32)

    out_ref[...] = acc


def kernel(x, shared_w1, shared_w3, shared_w2, routed_w1, routed_w3, routed_w2,
           gate_w, expert_bias):
    b, s, d = x.shape
    t = b * s
    f = shared_w1.shape[-1]
    n_routed = routed_w1.shape[0]
    xf = x.reshape(t, d)

    w1s = jnp.concatenate(list(shared_w1), axis=1)
    w3s = jnp.concatenate(list(shared_w3), axis=1)
    w2s = jnp.concatenate(list(shared_w2), axis=0)

    grid = (t // TB,)
    const = lambda i: (0, 0)
    const3 = lambda i: (0, 0, 0)

    out, idx = pl.pallas_call(
        functools.partial(_moe_block, n_routed=n_routed),
        grid=grid,
        in_specs=[
            pl.BlockSpec((TB, d), lambda i: (i, 0)),
            pl.BlockSpec(w1s.shape, const),
            pl.BlockSpec(w3s.shape, const),
            pl.BlockSpec(w2s.shape, const),
            pl.BlockSpec(routed_w1.shape, const3),
            pl.BlockSpec(routed_w3.shape, const3),
            pl.BlockSpec(routed_w2.shape, const3),
            pl.BlockSpec(gate_w.shape, const),
            pl.BlockSpec(expert_bias.shape, lambda i: (0,)),
        ],
        out_specs=[
            pl.BlockSpec((TB, d), lambda i: (i, 0)),
            pl.BlockSpec((TB, 2), lambda i: (i, 0)),
        ],
        out_shape=[
            jax.ShapeDtypeStruct((t, d), jnp.float32),
            jax.ShapeDtypeStruct((t, 2), jnp.int32),
        ],
    )(xf, w1s, w3s, w2s, routed_w1, routed_w3, routed_w2, gate_w, expert_bias)

    return out.reshape(b, s, d), idx.reshape(b, s, 2)
